# baseline (speedup 1.0000x reference)
import functools

import jax
import jax.numpy as jnp
from jax import lax
from jax.experimental import pallas as pl
from jax.experimental.pallas import tpu as pltpu
from jax.experimental.pallas import tpu_sc as plsc

B = 8
L = 1024
N = B * L
D = 32
V = 8192
BLK = 1024
NBLK = V // BLK
LB = 4096
NL = N // LB
BPT = LB // L
COMMIT = 0.02
EPS = 1e-12

_SC_CORES = 2
_SC_SUBCORES = 16
_SC_WORKERS = _SC_CORES * _SC_SUBCORES
_RPW = N // _SC_WORKERS
_GRP = _RPW // 16


def _vq_body(x_ref, w_ref, idx_ref, bs_ref, nx_ref, nw_ref, bs, bi, xn_s):
    nb = pl.program_id(1)

    for bb in range(BPT):
        xb = x_ref[bb]
        xnsq = jnp.sum(xb * xb, axis=0, keepdims=True)
        nx = jnp.maximum(jnp.sqrt(xnsq), EPS)
        nx_ref[:, pl.ds(bb * L, L)] = nx
        xn_s[:, pl.ds(bb * L, L)] = xb / nx

    w = w_ref[...]
    wnsq = jnp.sum(w * w, axis=1, keepdims=True)
    nw = jnp.maximum(jnp.sqrt(wnsq), EPS)
    nw_ref[...] = nw
    wn = w / nw

    s = lax.dot_general(wn, xn_s[...], (((1,), (0,)), ((), ())),
                        preferred_element_type=jnp.float32)

    smax = jnp.max(s, axis=0, keepdims=True)
    hit = s == smax
    rio = lax.broadcasted_iota(jnp.int32, (BLK, 1), 0).astype(jnp.float32)
    li = jnp.min(jnp.where(hit, rio, 1e9),
                 axis=0, keepdims=True)

    @pl.when(nb == 0)
    def _():
        bs[...] = jnp.full((1, LB), -jnp.inf, jnp.float32)
        bi[...] = jnp.zeros((1, LB), jnp.float32)

    upd = smax > bs[...]
    bs[...] = jnp.where(upd, smax, bs[...])
    bi[...] = jnp.where(upd, float(BLK) * nb + li, bi[...])

    @pl.when(nb == NBLK - 1)
    def _():
        idx_ref[...] = bi[...].astype(jnp.int32)
        bs_ref[...] = bs[...]


_vq_call = pl.pallas_call(
    _vq_body,
    grid=(NL, NBLK),
    in_specs=[
        pl.BlockSpec((BPT, D, L), lambda nl, nb: (nl, 0, 0)),
        pl.BlockSpec((BLK, D), lambda nl, nb: (nb, 0)),
    ],
    out_specs=[
        pl.BlockSpec((1, LB), lambda nl, nb: (0, nl)),
        pl.BlockSpec((1, LB), lambda nl, nb: (0, nl)),
        pl.BlockSpec((1, LB), lambda nl, nb: (0, nl)),
        pl.BlockSpec((BLK, 1), lambda nl, nb: (nb, 0)),
    ],
    out_shape=[
        jax.ShapeDtypeStruct((1, N), jnp.int32),
        jax.ShapeDtypeStruct((1, N), jnp.float32),
        jax.ShapeDtypeStruct((1, N), jnp.float32),
        jax.ShapeDtypeStruct((V, 1), jnp.float32),
    ],
    scratch_shapes=[
        pltpu.VMEM((1, LB), jnp.float32),
        pltpu.VMEM((1, LB), jnp.float32),
        pltpu.VMEM((D, LB), jnp.float32),
    ],
    compiler_params=pltpu.CompilerParams(
        dimension_semantics=("arbitrary", "arbitrary")),
)


@functools.cache
def _make_sc_gather_loss():
    @functools.partial(
        pl.kernel,
        mesh=plsc.VectorSubcoreMesh(core_axis_name="c", subcore_axis_name="s"),
        out_type=jax.ShapeDtypeStruct((N, D), jnp.float32),
        scratch_types=[
            pltpu.VMEM((_RPW,), jnp.int32),
            pltpu.VMEM((_RPW, D), jnp.float32),
            pltpu.SemaphoreType.DMA,
        ],
        compiler_params=pltpu.CompilerParams(use_tc_tiling_on_sc=False),
    )
    def _sc_gather(w_hbm, idx_hbm, out_hbm, idx_v, rows_v, sem):
        wid = lax.axis_index("s") * _SC_CORES + lax.axis_index("c")
        base = wid * _RPW
        pltpu.sync_copy(idx_hbm.at[pl.ds(base, _RPW)], idx_v)
        pltpu.async_copy(w_hbm.at[idx_v], rows_v, sem).wait()
        pltpu.sync_copy(rows_v, out_hbm.at[pl.ds(base, _RPW)])

    return _sc_gather


def _loss_body(q_ref, cos_ref, nx_ref, loss_ref, qt_ref, lacc):
    b = pl.program_id(0)
    qb = q_ref[...]
    qt_ref[0] = lax.transpose(qb, (1, 0))
    qnsq = jnp.sum(qb * qb, axis=1, keepdims=True)
    nw = jnp.maximum(jnp.sqrt(qnsq), EPS)
    cos = cos_ref[...]
    nx = nx_ref[...]
    crossvec = nx * cos
    cross = lax.dot_general(crossvec, nw, (((1,), (0,)), ((), ())),
                            preferred_element_type=jnp.float32)
    total = jnp.sum(qnsq) - 2.0 * cross[0, 0] + jnp.sum(nx * nx)

    @pl.when(b == 0)
    def _():
        lacc[0] = 0.0

    lacc[0] += total

    @pl.when(b == B - 1)
    def _():
        loss_ref[0, 0] = lacc[0] * ((1.0 + COMMIT) / (N * D))


_loss_call = pl.pallas_call(
    _loss_body,
    grid=(B,),
    in_specs=[
        pl.BlockSpec((L, D), lambda b: (b, 0)),
        pl.BlockSpec((1, L), lambda b: (0, b)),
        pl.BlockSpec((1, L), lambda b: (0, b)),
    ],
    out_specs=[
        pl.BlockSpec(memory_space=pltpu.SMEM),
        pl.BlockSpec((1, D, L), lambda b: (b, 0, 0)),
    ],
    out_shape=[
        jax.ShapeDtypeStruct((1, 1), jnp.float32),
        jax.ShapeDtypeStruct((B, D, L), jnp.float32),
    ],
    scratch_shapes=[pltpu.SMEM((1,), jnp.float32)],
    compiler_params=pltpu.CompilerParams(
        dimension_semantics=("arbitrary",)),
)


def kernel(inputs, weight):
    idx_row, cos_row, nx_row, _ = _vq_call(inputs, weight)
    idx_flat = idx_row.reshape(N)
    q = _make_sc_gather_loss()(weight, idx_flat)
    loss11, quantized_out = _loss_call(q, cos_row, nx_row)
    loss = loss11[0, 0]
    encoding_indices = idx_flat.reshape(N, 1)
    return (loss, quantized_out, encoding_indices)


# speedup vs baseline: 1.0321x; 1.0321x over previous
import functools

import jax
import jax.numpy as jnp
from jax import lax
from jax.experimental import pallas as pl
from jax.experimental.pallas import tpu as pltpu
from jax.experimental.pallas import tpu_sc as plsc

B = 8
L = 1024
N = B * L
D = 32
V = 8192
BLK = 1024
NBLK = V // BLK
LB = 8192
NL = N // LB
BPT = LB // L
COMMIT = 0.02
EPS = 1e-12

_SC_CORES = 2
_SC_SUBCORES = 16
_SC_WORKERS = _SC_CORES * _SC_SUBCORES
_RPW = N // _SC_WORKERS
_GRP = _RPW // 16


def _vq_body(x_ref, w_ref, idx_ref, bs_ref, nx_ref, bs, bi, xn_s):
    nb = pl.program_id(1)

    for bb in range(BPT):
        xb = x_ref[bb]
        xnsq = jnp.sum(xb * xb, axis=0, keepdims=True)
        nx = jnp.maximum(jnp.sqrt(xnsq), EPS)
        nx_ref[:, pl.ds(bb * L, L)] = nx
        xn_s[:, pl.ds(bb * L, L)] = xb / nx

    w = w_ref[...]
    wnsq = jnp.sum(w * w, axis=1, keepdims=True)
    nw = jnp.maximum(jnp.sqrt(wnsq), EPS)
    wn = w / nw

    s = lax.dot_general(wn, xn_s[...], (((1,), (0,)), ((), ())),
                        preferred_element_type=jnp.float32)

    smax = jnp.max(s, axis=0, keepdims=True)
    hit = s == smax
    rio = lax.broadcasted_iota(jnp.int32, (BLK, 1), 0).astype(jnp.float32)
    li = jnp.min(jnp.where(hit, rio, 1e9),
                 axis=0, keepdims=True)

    @pl.when(nb == 0)
    def _():
        bs[...] = jnp.full((1, LB), -jnp.inf, jnp.float32)
        bi[...] = jnp.zeros((1, LB), jnp.float32)

    upd = smax > bs[...]
    bs[...] = jnp.where(upd, smax, bs[...])
    bi[...] = jnp.where(upd, float(BLK) * nb + li, bi[...])

    @pl.when(nb == NBLK - 1)
    def _():
        idx_ref[...] = bi[...].astype(jnp.int32)
        bs_ref[...] = bs[...]


_vq_call = pl.pallas_call(
    _vq_body,
    grid=(NL, NBLK),
    in_specs=[
        pl.BlockSpec((BPT, D, L), lambda nl, nb: (nl, 0, 0)),
        pl.BlockSpec((BLK, D), lambda nl, nb: (nb, 0)),
    ],
    out_specs=[
        pl.BlockSpec((1, LB), lambda nl, nb: (0, nl)),
        pl.BlockSpec((1, LB), lambda nl, nb: (0, nl)),
        pl.BlockSpec((1, LB), lambda nl, nb: (0, nl)),
    ],
    out_shape=[
        jax.ShapeDtypeStruct((1, N), jnp.int32),
        jax.ShapeDtypeStruct((1, N), jnp.float32),
        jax.ShapeDtypeStruct((1, N), jnp.float32),
    ],
    scratch_shapes=[
        pltpu.VMEM((1, LB), jnp.float32),
        pltpu.VMEM((1, LB), jnp.float32),
        pltpu.VMEM((D, LB), jnp.float32),
    ],
    compiler_params=pltpu.CompilerParams(
        dimension_semantics=("arbitrary", "arbitrary")),
)


@functools.cache
def _make_sc_gather_loss():
    @functools.partial(
        pl.kernel,
        mesh=plsc.VectorSubcoreMesh(core_axis_name="c", subcore_axis_name="s"),
        out_type=jax.ShapeDtypeStruct((N, D), jnp.float32),
        scratch_types=[
            pltpu.VMEM((_RPW,), jnp.int32),
            pltpu.VMEM((_RPW, D), jnp.float32),
            pltpu.SemaphoreType.DMA,
        ],
        compiler_params=pltpu.CompilerParams(use_tc_tiling_on_sc=False),
    )
    def _sc_gather(w_hbm, idx_hbm, out_hbm, idx_v, rows_v, sem):
        wid = lax.axis_index("s") * _SC_CORES + lax.axis_index("c")
        base = wid * _RPW
        pltpu.sync_copy(idx_hbm.at[pl.ds(base, _RPW)], idx_v)
        pltpu.async_copy(w_hbm.at[idx_v], rows_v, sem).wait()
        pltpu.sync_copy(rows_v, out_hbm.at[pl.ds(base, _RPW)])

    return _sc_gather


def _loss_body(q_ref, cos_ref, nx_ref, loss_ref, qt_ref, lacc):
    b = pl.program_id(0)
    qb = q_ref[...]
    qt_ref[0] = lax.transpose(qb, (1, 0))
    qnsq = jnp.sum(qb * qb, axis=1, keepdims=True)
    nw = jnp.maximum(jnp.sqrt(qnsq), EPS)
    cos = cos_ref[...]
    nx = nx_ref[...]
    crossvec = nx * cos
    cross = lax.dot_general(crossvec, nw, (((1,), (0,)), ((), ())),
                            preferred_element_type=jnp.float32)
    total = jnp.sum(qnsq) - 2.0 * cross[0, 0] + jnp.sum(nx * nx)

    @pl.when(b == 0)
    def _():
        lacc[0] = 0.0

    lacc[0] += total

    @pl.when(b == B - 1)
    def _():
        loss_ref[0, 0] = lacc[0] * ((1.0 + COMMIT) / (N * D))


_loss_call = pl.pallas_call(
    _loss_body,
    grid=(B,),
    in_specs=[
        pl.BlockSpec((L, D), lambda b: (b, 0)),
        pl.BlockSpec((1, L), lambda b: (0, b)),
        pl.BlockSpec((1, L), lambda b: (0, b)),
    ],
    out_specs=[
        pl.BlockSpec(memory_space=pltpu.SMEM),
        pl.BlockSpec((1, D, L), lambda b: (b, 0, 0)),
    ],
    out_shape=[
        jax.ShapeDtypeStruct((1, 1), jnp.float32),
        jax.ShapeDtypeStruct((B, D, L), jnp.float32),
    ],
    scratch_shapes=[pltpu.SMEM((1,), jnp.float32)],
    compiler_params=pltpu.CompilerParams(
        dimension_semantics=("arbitrary",)),
)


def kernel(inputs, weight):
    idx_row, cos_row, nx_row = _vq_call(inputs, weight)
    idx_flat = idx_row.reshape(N)
    q = _make_sc_gather_loss()(weight, idx_flat)
    loss11, quantized_out = _loss_call(q, cos_row, nx_row)
    loss = loss11[0, 0]
    encoding_indices = idx_flat.reshape(N, 1)
    return (loss, quantized_out, encoding_indices)
